# baseline (speedup 1.0000x reference)
import functools

import jax
import jax.numpy as jnp
from jax import lax
from jax.experimental import pallas as pl
from jax.experimental.pallas import tpu as pltpu
from jax.experimental.pallas import tpu_sc as plsc

_N = 10000
_E = 160000
_H = 256
_POOL = 256
_G = 64
_BN_EPS = 1e-5
_EPS = 1e-7

_ROWS = 1000

_HC = 128
_NUNITS = 6
_NPAD = 10000
_CHUNK = 128
_NCHUNK = _E // _CHUNK
_TPW = 79


def _sc_seg_body(xq_hbm, eaq_hbm, src_hbm, dst_hbm, out_hbm,
                 acc, zbuf, src_v, dst_v, idx_v,
                 xrow_v, earow_v, valb, sem):
    cc = lax.axis_index("c")
    ss = lax.axis_index("s")

    def zinit(r, _):
        for c4 in range(8):
            zbuf[r, pl.ds(c4 * 16, 16)] = jnp.zeros((16,), jnp.float32)
        return 0
    lax.fori_loop(0, 8, zinit, 0)
    is_ex = cc == 0

    def pass_body(u, _):
        nk = jnp.where(ss == 15, 65, 79)

        def zpass(kk, _):
            pltpu.sync_copy(zbuf, acc.at[pl.ds(ss * 632 + kk * 8, 8)])
            return 0
        lax.fori_loop(0, nk, zpass, 0)
        plsc.subcore_barrier()

        def chunk_body(jj, _):
            cid = ss + 16 * jj

            @pl.when(cid < _NCHUNK)
            def _():
                base = cid * _CHUNK
                pltpu.sync_copy(src_hbm.at[pl.ds(base, _CHUNK)], src_v)
                pltpu.sync_copy(dst_hbm.at[pl.ds(base, _CHUNK)], dst_v)
                xoff = u * _N
                for k in range(8):
                    sl = pl.ds(k * 16, 16)
                    idx_v[sl] = src_v[sl] + xoff
                pltpu.sync_copy(eaq_hbm.at[pl.ds(u * _E + base, _CHUNK)],
                                earow_v)
                pltpu.async_copy(xq_hbm.at[idx_v], xrow_v, sem).wait()

                @pl.when(is_ex)
                def _():
                    def compe(k, _):
                        r0 = k * 8
                        for rr in range(8):
                            for c4 in range(8):
                                sl = pl.ds(c4 * 16, 16)
                                m = jnp.maximum(
                                    xrow_v[r0 + rr, sl] + earow_v[r0 + rr, sl],
                                    0.0) + _EPS
                                valb[r0 + rr, sl] = jnp.exp(m)
                        return 0
                    lax.fori_loop(0, _CHUNK // 8, compe, 0)

                @pl.when(jnp.logical_not(is_ex))
                def _():
                    def compm(k, _):
                        r0 = k * 8
                        for rr in range(8):
                            for c4 in range(8):
                                sl = pl.ds(c4 * 16, 16)
                                m = jnp.maximum(
                                    xrow_v[r0 + rr, sl] + earow_v[r0 + rr, sl],
                                    0.0) + _EPS
                                valb[r0 + rr, sl] = m * jnp.exp(m)
                        return 0
                    lax.fori_loop(0, _CHUNK // 8, compm, 0)
                pltpu.sync_copy(valb, acc.at[dst_v], add=True)
            return 0
        lax.fori_loop(0, _TPW, chunk_body, 0)
        plsc.subcore_barrier()
        out_off = (cc * _NUNITS + u) * _NPAD + ss * 632

        def wpass(kk, _):
            pltpu.sync_copy(acc.at[pl.ds(ss * 632 + kk * 8, 8)],
                            out_hbm.at[pl.ds(out_off + kk * 8, 8)])
            return 0
        lax.fori_loop(0, nk, wpass, 0)
        return 0
    lax.fori_loop(0, _NUNITS, pass_body, 0)


_sc_seg = pl.kernel(
    _sc_seg_body,
    out_type=jax.ShapeDtypeStruct((2 * _NUNITS * _NPAD, _HC), jnp.float32),
    mesh=plsc.VectorSubcoreMesh(core_axis_name="c", subcore_axis_name="s"),
    scratch_types=[
        pltpu.VMEM_SHARED((_NPAD, _HC), jnp.float32),
        pltpu.VMEM((8, _HC), jnp.float32),
        pltpu.VMEM((_CHUNK,), jnp.int32),
        pltpu.VMEM((_CHUNK,), jnp.int32),
        pltpu.VMEM((_CHUNK,), jnp.int32),
        pltpu.VMEM((_CHUNK, _HC), jnp.float32),
        pltpu.VMEM((_CHUNK, _HC), jnp.float32),
        pltpu.VMEM((_CHUNK, _HC), jnp.float32),
        pltpu.SemaphoreType.DMA,
    ],
)


def _layer_dense_body(agg_ref, x_ref, w1_ref, b1_ref, g1_ref, bb1_ref,
                      w2_ref, b2_ref, ms_ref, lng_ref, lnb_ref, out_ref):
    agg = agg_ref[...]
    x = x_ref[...]
    agg_norm = jnp.sqrt(jnp.sum(agg * agg, axis=-1, keepdims=True))
    agg_n = agg / jnp.maximum(agg_norm, 1e-12)
    x_norm = jnp.sqrt(jnp.sum(x * x, axis=-1, keepdims=True))
    out = agg_n * x_norm * ms_ref[0] + x
    h = jnp.dot(out, w1_ref[...], preferred_element_type=jnp.float32) + b1_ref[...]
    h = (h / jnp.sqrt(1.0 + _BN_EPS)) * g1_ref[...] + bb1_ref[...]
    h = jnp.maximum(h, 0.0)
    h = jnp.dot(h, w2_ref[...], preferred_element_type=jnp.float32) + b2_ref[...]
    mu = jnp.mean(h, axis=-1, keepdims=True)
    var = jnp.mean((h - mu) ** 2, axis=-1, keepdims=True)
    h = (h - mu) / jnp.sqrt(var + 1e-5) * lng_ref[...] + lnb_ref[...]
    h = jnp.maximum(h, 0.0)
    m = jnp.max(h, axis=-1, keepdims=True)
    e = jnp.exp(h - m)
    out_ref[...] = e / jnp.sum(e, axis=-1, keepdims=True)


def _layer_dense(agg, x, w1, b1, g1, bb1, w2, b2, ms, lng, lnb):
    grid = (_N // _ROWS,)
    rowspec = pl.BlockSpec((_ROWS, _H), lambda i: (i, 0))
    full = lambda a: pl.BlockSpec(a.shape, lambda i: (0,) * a.ndim)
    return pl.pallas_call(
        _layer_dense_body,
        grid=grid,
        in_specs=[rowspec, rowspec,
                  full(w1), full(b1), full(g1), full(bb1),
                  full(w2), full(b2),
                  pl.BlockSpec(memory_space=pltpu.SMEM),
                  full(lng), full(lnb)],
        out_specs=pl.BlockSpec((_ROWS, _POOL), lambda i: (i, 0)),
        out_shape=jax.ShapeDtypeStruct((_N, _POOL), jnp.float32),
    )(agg, x, w1, b1, g1, bb1, w2, b2, ms, lng, lnb)


def _regressor_body(p_ref, w1_ref, b1_ref, g1_ref, bb1_ref,
                    w2_ref, b2_ref, g2_ref, bb2_ref, w3_ref, b3_ref, out_ref):
    z = jnp.dot(p_ref[...], w1_ref[...], preferred_element_type=jnp.float32) + b1_ref[...]
    z = jnp.maximum(z, 0.0)
    z = (z / jnp.sqrt(1.0 + _BN_EPS)) * g1_ref[...] + bb1_ref[...]
    z = jnp.dot(z, w2_ref[...], preferred_element_type=jnp.float32) + b2_ref[...]
    z = jnp.maximum(z, 0.0)
    z = (z / jnp.sqrt(1.0 + _BN_EPS)) * g2_ref[...] + bb2_ref[...]
    z = jnp.dot(z, w3_ref[...], preferred_element_type=jnp.float32) + b3_ref[...]
    out_ref[...] = z


def _regressor(pooled, w1, b1, g1, bb1, w2, b2, g2, bb2, w3, b3):
    args = (pooled, w1, b1, g1, bb1, w2, b2, g2, bb2, w3, b3)
    return pl.pallas_call(
        _regressor_body,
        in_specs=[pl.BlockSpec(a.shape, functools.partial(lambda n: (0,) * n, a.ndim))
                  for a in args],
        out_specs=pl.BlockSpec((_G, 1), lambda: (0, 0)),
        out_shape=jax.ShapeDtypeStruct((_G, 1), jnp.float32),
    )(*args)


def kernel(atom_feat, edge_attr, node_W, node_b, edge_W, edge_b, mlp_W1, mlp_b1,
           bn1_g, bn1_b, mlp_W2, mlp_b2, msg_scale, ln_g, ln_b, reg_W1, reg_b1,
           rbn1_g, rbn1_b, reg_W2, reg_b2, rbn2_g, rbn2_b, reg_W3, reg_b3,
           edge_index, batch):
    src = edge_index[0]
    dst = edge_index[1]
    xq = jnp.einsum('nd,ldhc->lhnc', atom_feat,
                    node_W.reshape(3, 256, 2, _HC)) \
        + node_b.reshape(3, 2, 1, _HC)
    eaq = jnp.einsum('ed,ldhc->lhec', edge_attr,
                     edge_W.reshape(3, 16, 2, _HC)) \
        + edge_b.reshape(3, 2, 1, _HC)
    x_full = xq.transpose(0, 2, 1, 3).reshape(3, _N, 256)
    sums = _sc_seg(xq.reshape(_NUNITS * _N, _HC),
                   eaq.reshape(_NUNITS * _E, _HC), src, dst)
    sums = sums.reshape(2, 3, 2, _NPAD, _HC)[:, :, :, :_N, :]
    readout = jnp.zeros((_N, _POOL), dtype=jnp.float32)
    for i in range(3):
        x = x_full[i]
        s_ex = sums[0, i].transpose(1, 0, 2).reshape(_N, 256)
        s_mex = sums[1, i].transpose(1, 0, 2).reshape(_N, 256)
        agg = s_mex / (s_ex + 1e-16)
        readout = readout + _layer_dense(agg, x, mlp_W1[i], mlp_b1[i], bn1_g[i],
                                         bn1_b[i], mlp_W2[i], mlp_b2[i],
                                         msg_scale[i:i + 1], ln_g[i], ln_b[i])
    pooled = jax.ops.segment_sum(readout, batch, num_segments=_G)
    return _regressor(pooled, reg_W1, reg_b1, rbn1_g, rbn1_b,
                      reg_W2, reg_b2, rbn2_g, rbn2_b, reg_W3, reg_b3)


# speedup vs baseline: 1.1608x; 1.1608x over previous
import functools

import jax
import jax.numpy as jnp
from jax import lax
from jax.experimental import pallas as pl
from jax.experimental.pallas import tpu as pltpu
from jax.experimental.pallas import tpu_sc as plsc

_N = 10000
_E = 160000
_H = 256
_POOL = 256
_G = 64
_BN_EPS = 1e-5
_EPS = 1e-7

_ROWS = 1000

_HC = 128
_NUNITS = 6
_NPAD = 10000
_CHUNK = 128
_NCHUNK = _E // _CHUNK
_TPW = 79


def _sc_seg_body(xq_hbm, eaq_hbm, src_hbm, dst_hbm, out_hbm,
                 acc, zbuf, src_v, dst_v, idx_v,
                 xrow_v, earow_v, valb, sem):
    cc = lax.axis_index("c")
    ss = lax.axis_index("s")

    def zinit(r, _):
        for c4 in range(8):
            zbuf[r, pl.ds(c4 * 16, 16)] = jnp.zeros((16,), jnp.float32)
        return 0
    lax.fori_loop(0, 8, zinit, 0)
    is_ex = cc == 0

    def pass_body(u, _):
        nk = jnp.where(ss == 15, 65, 79)

        def zpass(kk, _):
            pltpu.sync_copy(zbuf, acc.at[pl.ds(ss * 632 + kk * 8, 8)])
            return 0
        lax.fori_loop(0, nk, zpass, 0)
        plsc.subcore_barrier()

        def chunk_body(jj, _):
            cid = ss + 16 * jj

            @pl.when(cid < _NCHUNK)
            def _():
                base = cid * _CHUNK
                pltpu.sync_copy(src_hbm.at[pl.ds(base, _CHUNK)], src_v)
                xoff = u * _N
                for k in range(8):
                    sl = pl.ds(k * 16, 16)
                    idx_v[sl] = src_v[sl] + xoff
                hg = pltpu.async_copy(xq_hbm.at[idx_v], xrow_v, sem)
                pltpu.sync_copy(dst_hbm.at[pl.ds(base, _CHUNK)], dst_v)
                pltpu.sync_copy(eaq_hbm.at[pl.ds(u * _E + base, _CHUNK)],
                                earow_v)
                hg.wait()

                @pl.when(is_ex)
                def _():
                    def compe(k, _):
                        r0 = k * 8
                        for rr in range(8):
                            for c4 in range(8):
                                sl = pl.ds(c4 * 16, 16)
                                m = jnp.maximum(
                                    xrow_v[r0 + rr, sl] + earow_v[r0 + rr, sl],
                                    0.0) + _EPS
                                valb[r0 + rr, sl] = jnp.exp(m)
                        return 0
                    lax.fori_loop(0, _CHUNK // 8, compe, 0)

                @pl.when(jnp.logical_not(is_ex))
                def _():
                    def compm(k, _):
                        r0 = k * 8
                        for rr in range(8):
                            for c4 in range(8):
                                sl = pl.ds(c4 * 16, 16)
                                m = jnp.maximum(
                                    xrow_v[r0 + rr, sl] + earow_v[r0 + rr, sl],
                                    0.0) + _EPS
                                valb[r0 + rr, sl] = m * jnp.exp(m)
                        return 0
                    lax.fori_loop(0, _CHUNK // 8, compm, 0)
                pltpu.sync_copy(valb, acc.at[dst_v], add=True)
            return 0
        lax.fori_loop(0, _TPW, chunk_body, 0)
        plsc.subcore_barrier()
        out_off = (cc * _NUNITS + u) * _NPAD + ss * 632

        def wpass(kk, _):
            pltpu.sync_copy(acc.at[pl.ds(ss * 632 + kk * 8, 8)],
                            out_hbm.at[pl.ds(out_off + kk * 8, 8)])
            return 0
        lax.fori_loop(0, nk, wpass, 0)
        return 0
    lax.fori_loop(0, _NUNITS, pass_body, 0)


_sc_seg = pl.kernel(
    _sc_seg_body,
    out_type=jax.ShapeDtypeStruct((2 * _NUNITS * _NPAD, _HC), jnp.float32),
    mesh=plsc.VectorSubcoreMesh(core_axis_name="c", subcore_axis_name="s"),
    scratch_types=[
        pltpu.VMEM_SHARED((_NPAD, _HC), jnp.float32),
        pltpu.VMEM((8, _HC), jnp.float32),
        pltpu.VMEM((_CHUNK,), jnp.int32),
        pltpu.VMEM((_CHUNK,), jnp.int32),
        pltpu.VMEM((_CHUNK,), jnp.int32),
        pltpu.VMEM((_CHUNK, _HC), jnp.float32),
        pltpu.VMEM((_CHUNK, _HC), jnp.float32),
        pltpu.VMEM((_CHUNK, _HC), jnp.float32),
        pltpu.SemaphoreType.DMA,
    ],
)


def _layer_dense_body(agg_ref, x_ref, w1_ref, b1_ref, g1_ref, bb1_ref,
                      w2_ref, b2_ref, ms_ref, lng_ref, lnb_ref, out_ref):
    agg = agg_ref[...]
    x = x_ref[...]
    agg_norm = jnp.sqrt(jnp.sum(agg * agg, axis=-1, keepdims=True))
    agg_n = agg / jnp.maximum(agg_norm, 1e-12)
    x_norm = jnp.sqrt(jnp.sum(x * x, axis=-1, keepdims=True))
    out = agg_n * x_norm * ms_ref[0] + x
    h = jnp.dot(out, w1_ref[...], preferred_element_type=jnp.float32) + b1_ref[...]
    h = (h / jnp.sqrt(1.0 + _BN_EPS)) * g1_ref[...] + bb1_ref[...]
    h = jnp.maximum(h, 0.0)
    h = jnp.dot(h, w2_ref[...], preferred_element_type=jnp.float32) + b2_ref[...]
    mu = jnp.mean(h, axis=-1, keepdims=True)
    var = jnp.mean((h - mu) ** 2, axis=-1, keepdims=True)
    h = (h - mu) / jnp.sqrt(var + 1e-5) * lng_ref[...] + lnb_ref[...]
    h = jnp.maximum(h, 0.0)
    m = jnp.max(h, axis=-1, keepdims=True)
    e = jnp.exp(h - m)
    out_ref[...] = e / jnp.sum(e, axis=-1, keepdims=True)


def _layer_dense(agg, x, w1, b1, g1, bb1, w2, b2, ms, lng, lnb):
    grid = (_N // _ROWS,)
    rowspec = pl.BlockSpec((_ROWS, _H), lambda i: (i, 0))
    full = lambda a: pl.BlockSpec(a.shape, lambda i: (0,) * a.ndim)
    return pl.pallas_call(
        _layer_dense_body,
        grid=grid,
        in_specs=[rowspec, rowspec,
                  full(w1), full(b1), full(g1), full(bb1),
                  full(w2), full(b2),
                  pl.BlockSpec(memory_space=pltpu.SMEM),
                  full(lng), full(lnb)],
        out_specs=pl.BlockSpec((_ROWS, _POOL), lambda i: (i, 0)),
        out_shape=jax.ShapeDtypeStruct((_N, _POOL), jnp.float32),
    )(agg, x, w1, b1, g1, bb1, w2, b2, ms, lng, lnb)


def _regressor_body(p_ref, w1_ref, b1_ref, g1_ref, bb1_ref,
                    w2_ref, b2_ref, g2_ref, bb2_ref, w3_ref, b3_ref, out_ref):
    z = jnp.dot(p_ref[...], w1_ref[...], preferred_element_type=jnp.float32) + b1_ref[...]
    z = jnp.maximum(z, 0.0)
    z = (z / jnp.sqrt(1.0 + _BN_EPS)) * g1_ref[...] + bb1_ref[...]
    z = jnp.dot(z, w2_ref[...], preferred_element_type=jnp.float32) + b2_ref[...]
    z = jnp.maximum(z, 0.0)
    z = (z / jnp.sqrt(1.0 + _BN_EPS)) * g2_ref[...] + bb2_ref[...]
    z = jnp.dot(z, w3_ref[...], preferred_element_type=jnp.float32) + b3_ref[...]
    out_ref[...] = z


def _regressor(pooled, w1, b1, g1, bb1, w2, b2, g2, bb2, w3, b3):
    args = (pooled, w1, b1, g1, bb1, w2, b2, g2, bb2, w3, b3)
    return pl.pallas_call(
        _regressor_body,
        in_specs=[pl.BlockSpec(a.shape, functools.partial(lambda n: (0,) * n, a.ndim))
                  for a in args],
        out_specs=pl.BlockSpec((_G, 1), lambda: (0, 0)),
        out_shape=jax.ShapeDtypeStruct((_G, 1), jnp.float32),
    )(*args)


def kernel(atom_feat, edge_attr, node_W, node_b, edge_W, edge_b, mlp_W1, mlp_b1,
           bn1_g, bn1_b, mlp_W2, mlp_b2, msg_scale, ln_g, ln_b, reg_W1, reg_b1,
           rbn1_g, rbn1_b, reg_W2, reg_b2, rbn2_g, rbn2_b, reg_W3, reg_b3,
           edge_index, batch):
    src = edge_index[0]
    dst = edge_index[1]
    xq = jnp.einsum('nd,ldhc->lhnc', atom_feat,
                    node_W.reshape(3, 256, 2, _HC)) \
        + node_b.reshape(3, 2, 1, _HC)
    eaq = jnp.einsum('ed,ldhc->lhec', edge_attr,
                     edge_W.reshape(3, 16, 2, _HC)) \
        + edge_b.reshape(3, 2, 1, _HC)
    x_full = xq.transpose(0, 2, 1, 3).reshape(3, _N, 256)
    sums = _sc_seg(xq.reshape(_NUNITS * _N, _HC),
                   eaq.reshape(_NUNITS * _E, _HC), src, dst)
    sums = sums.reshape(2, 3, 2, _NPAD, _HC)[:, :, :, :_N, :]
    readout = jnp.zeros((_N, _POOL), dtype=jnp.float32)
    for i in range(3):
        x = x_full[i]
        s_ex = sums[0, i].transpose(1, 0, 2).reshape(_N, 256)
        s_mex = sums[1, i].transpose(1, 0, 2).reshape(_N, 256)
        agg = s_mex / (s_ex + 1e-16)
        readout = readout + _layer_dense(agg, x, mlp_W1[i], mlp_b1[i], bn1_g[i],
                                         bn1_b[i], mlp_W2[i], mlp_b2[i],
                                         msg_scale[i:i + 1], ln_g[i], ln_b[i])
    pooled = jax.ops.segment_sum(readout, batch, num_segments=_G)
    return _regressor(pooled, reg_W1, reg_b1, rbn1_g, rbn1_b,
                      reg_W2, reg_b2, rbn2_g, rbn2_b, reg_W3, reg_b3)
